# baseline (speedup 1.0000x reference)
import jax
import jax.numpy as jnp
from jax import lax
from jax.experimental import pallas as pl
from jax.experimental.pallas import tpu as pltpu
from jax.experimental.pallas import tpu_sc as plsc

_T = 16
_C = 96
_H = 224
_W = 224
_K = 8
_HW = _H * _W
_TC = _T * _C
_CTILE = 128
_NCHUNK = _TC // _CTILE
_GC = 48


def _two_sum(a, b):
    s = a + b
    bb = s - a
    err = (a - (s - bb)) + (b - bb)
    return s, err


def _split(a):
    c = a * 4097.0
    hi = c - (c - a)
    return hi, a - hi


def _two_prod(a, b):
    p = a * b
    ah, al = _split(a)
    bh, bl = _split(b)
    err = ((ah * bh - p) + ah * bl + al * bh) + al * bl
    return p, err


def _ff_add(ah, al, bh, bl):
    sh, se = _two_sum(ah, bh)
    se = se + (al + bl)
    hi = sh + se
    lo = se - (hi - sh)
    return hi, lo


def _pool_kernel(x_ref, sum_ref, max_ref):
    blk = x_ref[...]
    sum_ref[...] = jnp.sum(blk, axis=(2, 3), keepdims=True)
    max_ref[...] = jnp.max(blk, axis=(2, 3), keepdims=True)


def _rank_kernel(sum_ref, max_ref, wt_ref, b_ref, idx_ref):
    acc_h = jnp.zeros((1, _T), jnp.float32)
    acc_l = jnp.zeros((1, _T), jnp.float32)
    for i in range(_NCHUNK):
        sl_ = slice(i * _CTILE, (i + 1) * _CTILE)
        avg = sum_ref[sl_, :] * (1.0 / _HW)
        mx = max_ref[sl_, :]
        sh, sl2 = _two_sum(avg, mx)
        sh_b = jnp.broadcast_to(sh, (_CTILE, _T))
        sl_b = jnp.broadcast_to(sl2, (_CTILE, _T))
        wt = wt_ref[sl_, :]
        ph, plo = _two_prod(wt, sh_b)
        plo = plo + wt * sl_b
        n = _CTILE
        while n > 1:
            n //= 2
            ph, plo = _ff_add(ph[:n], plo[:n], ph[n:], plo[n:])
        acc_h, acc_l = _ff_add(acc_h, acc_l, ph, plo)

    bh, bl = _ff_add(acc_h, acc_l, 2.0 * b_ref[...],
                     jnp.zeros((1, _T), jnp.float32))
    idx_ref[...] = bh + bl


def _sc_topk_kernel(logits_hbm, idx_hbm, lvm, ivm):
    cid = lax.axis_index("c")
    sid = lax.axis_index("s")

    pltpu.sync_copy(logits_hbm, lvm)
    l = lvm[...]
    iota = lax.iota(jnp.int32, _T)
    k1, v1 = plsc.sort_key_val(l, iota, descending=True)
    prev = lax.gather(
        k1, jnp.maximum(iota - 1, 0).reshape(_T, 1),
        lax.GatherDimensionNumbers(offset_dims=(), collapsed_slice_dims=(0,),
                                   start_index_map=(0,)),
        slice_sizes=(1,),
        mode=lax.GatherScatterMode.PROMISE_IN_BOUNDS)
    is_new = jnp.logical_or(k1 != prev, iota == 0).astype(jnp.int32)
    run = jnp.cumsum(is_new)
    _, v2 = plsc.sort_key_val(run * _T + v1, v1, descending=False)
    ivm[...] = v2

    @pl.when(jnp.logical_and(cid == 0, sid == 0))
    def _():
        pltpu.sync_copy(ivm.at[pl.ds(0, _K)], idx_hbm)


def _gather_kernel(idx_ref, x_ref, out_ref):
    out_ref[...] = x_ref[...]


def kernel(x, W_fc, b_fc, k):
    del k
    sums, maxes = pl.pallas_call(
        _pool_kernel,
        grid=(_T,),
        in_specs=[pl.BlockSpec((1, _C, _H, _W), lambda t: (t, 0, 0, 0))],
        out_specs=[
            pl.BlockSpec((1, _C, 1, 1), lambda t: (t, 0, 0, 0)),
            pl.BlockSpec((1, _C, 1, 1), lambda t: (t, 0, 0, 0)),
        ],
        out_shape=[
            jax.ShapeDtypeStruct((_T, _C, 1, 1), jnp.float32),
            jax.ShapeDtypeStruct((_T, _C, 1, 1), jnp.float32),
        ],
    )(x)

    sum_col = sums.reshape(_TC, 1)
    max_col = maxes.reshape(_TC, 1)
    wt = W_fc.T
    b_row = b_fc.reshape(1, _T)

    logits = pl.pallas_call(
        _rank_kernel,
        out_shape=jax.ShapeDtypeStruct((1, _T), jnp.float32),
    )(sum_col, max_col, wt, b_row)

    mesh = plsc.VectorSubcoreMesh(core_axis_name="c", subcore_axis_name="s")
    idx = pl.kernel(
        _sc_topk_kernel,
        mesh=mesh,
        out_type=jax.ShapeDtypeStruct((_K,), jnp.int32),
        scratch_types=[
            pltpu.VMEM((_T,), jnp.float32),
            pltpu.VMEM((_T,), jnp.int32),
        ],
        compiler_params=pltpu.CompilerParams(needs_layout_passes=False),
    )(logits.reshape(_T))

    out = pl.pallas_call(
        _gather_kernel,
        grid_spec=pltpu.PrefetchScalarGridSpec(
            num_scalar_prefetch=1,
            grid=(_K, _C // _GC),
            in_specs=[
                pl.BlockSpec((1, _GC, _H, _W), lambda kk, c, idx_ref: (idx_ref[kk], c, 0, 0)),
            ],
            out_specs=pl.BlockSpec((1, _GC, _H, _W), lambda kk, c, idx_ref: (kk, c, 0, 0)),
        ),
        out_shape=jax.ShapeDtypeStruct((_K, _C, _H, _W), jnp.float32),
    )(idx, x)
    return out
